# baseline (speedup 1.0000x reference)
import functools

import jax
import jax.numpy as jnp
from jax import lax
from jax.experimental import pallas as pl
from jax.experimental.pallas import tpu as pltpu
from jax.experimental.pallas import tpu_sc as plsc

NC = 2
NS = 16
L = 16
NW = NC * NS

B = 16384
D = 16
CHUNK = B // NW
GCH = 128
NG = CHUNK // GCH


_mesh = plsc.VectorSubcoreMesh(core_axis_name="c", subcore_axis_name="s")


@functools.partial(
    pl.kernel,
    out_type=jax.ShapeDtypeStruct((B,), jnp.float32),
    mesh=_mesh,
    compiler_params=pltpu.CompilerParams(
        needs_layout_passes=False, use_tc_tiling_on_sc=False),
    scratch_types=[
        pltpu.VMEM((CHUNK,), jnp.int32),
        pltpu.VMEM((CHUNK,), jnp.int32),
        pltpu.VMEM((CHUNK, D), jnp.float32),
        pltpu.VMEM((CHUNK, D), jnp.float32),
        pltpu.VMEM((CHUNK,), jnp.float32),
        pltpu.VMEM((CHUNK,), jnp.float32),
        pltpu.VMEM((CHUNK,), jnp.float32),
        pltpu.SemaphoreType.DMA,
    ],
)
def _embed(i_hbm, j_hbm, wi_hbm, wj_hbm, bi_hbm, bj_hbm, out_hbm,
           idx_i, idx_j, wi_rows, wj_rows, bi_v, bj_v, out_v, sem):
    wid = lax.axis_index("c") * NS + lax.axis_index("s")
    base = wid * CHUNK

    pltpu.sync_copy(i_hbm.at[pl.ds(base, CHUNK)], idx_i)
    pltpu.sync_copy(j_hbm.at[pl.ds(base, CHUNK)], idx_j)

    copies = []
    for c in range(NG):
        sl = pl.ds(c * GCH, GCH)
        copies.append(pltpu.async_copy(wi_hbm.at[idx_i.at[sl]], wi_rows.at[sl], sem))
        copies.append(pltpu.async_copy(wj_hbm.at[idx_j.at[sl]], wj_rows.at[sl], sem))
        copies.append(pltpu.async_copy(bi_hbm.at[idx_i.at[sl]], bi_v.at[sl], sem))
        copies.append(pltpu.async_copy(bj_hbm.at[idx_i.at[sl]], bj_v.at[sl], sem))
    for cp in copies:
        cp.wait()

    lane = lax.iota(jnp.int32, L)

    def group(g, carry):
        start = pl.multiple_of(g * L, L)
        row = g * L + lane
        acc = bi_v[pl.ds(start, L)] + bj_v[pl.ds(start, L)]
        for d in range(D):
            col = jnp.full((L,), d, jnp.int32)
            acc = acc + (plsc.load_gather(wi_rows, [row, col])
                         * plsc.load_gather(wj_rows, [row, col]))
        out_v[pl.ds(start, L)] = acc
        return carry

    lax.fori_loop(0, CHUNK // L, group, 0)

    pltpu.sync_copy(out_v, out_hbm.at[pl.ds(base, CHUNK)])


@jax.jit
def kernel(i, j, wi, wj, bi, bj):
    return _embed(i.astype(jnp.int32), j.astype(jnp.int32),
                  wi, wj, bi.reshape(-1), bj.reshape(-1))
